# baseline (speedup 1.0000x reference)
import functools

import numpy as np
import jax
import jax.numpy as jnp
from jax import lax
from jax.experimental import pallas as pl
from jax.experimental.pallas import tpu as pltpu
from jax.experimental.pallas import tpu_sc as plsc

L = 16
B_DIM, Q_DIM, N_DIM = 2, 4096, 4096
R_TOTAL = B_DIM * Q_DIM
K_TOP = 2048
NUM_WORKERS = 32
HALF = R_TOTAL // (2 * NUM_WORKERS)
INT_MIN = np.int32(-2147483648)
ALL_ONES = np.int32(-1)


def _splat(x):
    return jnp.full((L,), x, jnp.int32)


def _sc_sort_kernel(x_hbm, idx_hbm, kthr_hbm, jthr_hbm,
                    xb0, xb1, ka, ia, kb, ib,
                    hall0, hall1, r0, r1, r2, r3,
                    thrk, thrj, sem0, sem1, semo):
    wid = lax.axis_index("s") * 2 + lax.axis_index("c")
    lanes = lax.iota(jnp.int32, L)
    ones = _splat(1)
    runs = [r0, r1, r2, r3]

    def scan4(h_all, h_zero):
        zeros = jnp.zeros((L,), jnp.int32)
        def chunk(t, carry):
            sl = pl.ds(t * L, L)
            a = h_all[pl.ds(t * L, L)]
            b = h_all[pl.ds(256 + t * L, L)]
            c = h_all[pl.ds(512 + t * L, L)]
            d = h_all[pl.ds(768 + t * L, L)]
            ab = a + b
            abc = ab + c
            tot = abc + d
            incl = plsc.cumsum(tot)
            excl = incl - tot + carry
            r0[sl] = excl
            r1[sl] = excl + a
            r2[sl] = excl + ab
            r3[sl] = excl + abc
            h_zero[pl.ds(t * L, L)] = zeros
            h_zero[pl.ds(256 + t * L, L)] = zeros
            h_zero[pl.ds(512 + t * L, L)] = zeros
            h_zero[pl.ds(768 + t * L, L)] = zeros
            return carry + incl[L - 1]
        lax.fori_loop(0, 256 // L, chunk, jnp.int32(0))

    def zero_all(h_all):
        def chunk(t, c):
            h_all[pl.ds(t * L, L)] = jnp.zeros((L,), jnp.int32)
            return c
        lax.fori_loop(0, 1024 // L, chunk, jnp.int32(0))

    def hist_pass(vq, src, shift, h_all):
        def one(t, woff):
            kv = src[pl.ds(t * L, L)]
            if shift == 24:
                d = lax.shift_right_logical(kv, _splat(24))
            else:
                d = lax.bitwise_and(
                    lax.shift_right_logical(kv, _splat(shift)), _splat(255))
            plsc.addupdate_scatter(h_all, [d + _splat(woff)], ones)
        def body(u, c):
            one(u, 0)
            one(vq + u, 256)
            one(2 * vq + u, 512)
            one(3 * vq + u, 768)
            return c
        lax.fori_loop(0, vq, body, jnp.int32(0))

    def radix_pass(vq, shift, ksrc, isrc, kdst, idst):
        def one(t, run):
            kv = ksrc[pl.ds(t * L, L)]
            if shift == 0:
                d = lax.bitwise_and(kv, _splat(255))
            elif shift == 24:
                d = lax.shift_right_logical(kv, _splat(24))
            else:
                d = lax.bitwise_and(
                    lax.shift_right_logical(kv, _splat(shift)), _splat(255))
            occ, _ = plsc.scan_count(d)
            base = plsc.load_gather(run, [d])
            pos = base + occ - ones
            if isrc is None:
                iv = lanes + t * L
            else:
                iv = isrc[pl.ds(t * L, L)]
            plsc.store_scatter(kdst, [pos], kv)
            plsc.store_scatter(idst, [pos], iv)
            plsc.store_scatter(run, [d], base + occ)
        def body(u, c):
            one(u, r0)
            one(vq + u, r1)
            one(2 * vq + u, r2)
            one(3 * vq + u, r3)
            return c
        lax.fori_loop(0, vq, body, jnp.int32(0))

    def row_of(g):
        row_a = wid * HALF + g
        row_b = R_TOTAL - 1 - wid * HALF - (g - HALF)
        return jnp.where(g < HALF, row_a, row_b)

    def row_body(i, xb, sem_self, xb_other, sem_other):
        row = row_of(i)
        q = lax.rem(row, jnp.int32(Q_DIM))
        vq = (q + 4 * L) // (4 * L)
        v = 4 * vq

        nxt = row_of(lax.min(i + 1, jnp.int32(2 * HALF - 1)))
        pltpu.async_copy(x_hbm.at[nxt], xb_other, sem_other)
        pltpu.make_async_copy(x_hbm.at[row], xb, sem_self).wait()

        def build1(t, woff):
            xv = xb[pl.ds(t * L, L)]
            bv = plsc.bitcast(xv, jnp.int32)
            key = jnp.where(bv >= 0, bv, INT_MIN - bv)
            inv = lax.bitwise_xor(key, _splat(0x7FFFFFFF))
            jv = lanes + t * L
            inv = jnp.where(jv <= q, inv, _splat(ALL_ONES))
            ka[pl.ds(t * L, L)] = inv
            d0 = lax.bitwise_and(inv, _splat(255))
            plsc.addupdate_scatter(hall0, [d0 + _splat(woff)], ones)

        def build(u, c2):
            build1(u, 0)
            build1(vq + u, 256)
            build1(2 * vq + u, 512)
            build1(3 * vq + u, 768)
            return c2
        lax.fori_loop(0, vq, build, jnp.int32(0))

        scan4(hall0, hall1)
        radix_pass(vq, 0, ka, None, kb, ib)
        hist_pass(vq, kb, 8, hall1)
        pltpu.make_async_copy(ia.at[pl.ds(0, K_TOP)], idx_hbm.at[row],
                              semo).wait()
        scan4(hall1, hall0)
        radix_pass(vq, 8, kb, ib, ka, ia)
        hist_pass(vq, ka, 16, hall0)
        scan4(hall0, hall1)
        radix_pass(vq, 16, ka, ia, kb, ib)
        hist_pass(vq, kb, 24, hall1)
        scan4(hall1, hall0)
        radix_pass(vq, 24, kb, ib, ka, ia)

        def fill(t, c2):
            ia[pl.ds(t * L, L)] = lanes + t * L
            return c2
        lax.fori_loop(lax.min(v, jnp.int32(K_TOP // L)),
                      jnp.int32(K_TOP // L), fill, jnp.int32(0))

        has = (v * L) >= K_TOP
        tvk = ka[pl.ds(K_TOP - L, L)]
        tvj = ia[pl.ds(K_TOP - L, L)]
        ithr = jnp.where(has, tvk[L - 1], ALL_ONES)
        jthr = jnp.where(has, tvj[L - 1], jnp.int32(K_TOP - 1))
        kthr = lax.bitwise_xor(ithr, jnp.int32(0x7FFFFFFF))
        pos = _splat(jnp.where(i < HALF, i, 3 * HALF - 1 - i))
        plsc.store_scatter(thrk, [pos], _splat(kthr))
        plsc.store_scatter(thrj, [pos], _splat(jthr))

        pltpu.async_copy(ia.at[pl.ds(0, K_TOP)], idx_hbm.at[row], semo)

    row0 = row_of(jnp.int32(0))
    pltpu.async_copy(x_hbm.at[row0], xb0, sem0)
    pltpu.async_copy(ia.at[pl.ds(0, K_TOP)], idx_hbm.at[row0], semo)
    zero_all(hall0)

    def pair_body(p, c):
        row_body(2 * p, xb0, sem0, xb1, sem1)
        row_body(2 * p + 1, xb1, sem1, xb0, sem0)
        return c

    lax.fori_loop(0, HALF, pair_body, jnp.int32(0))

    pltpu.make_async_copy(ia.at[pl.ds(0, K_TOP)], idx_hbm.at[row0],
                          semo).wait()
    pltpu.make_async_copy(x_hbm.at[row0], xb0, sem0).wait()

    a0 = wid * HALF
    b0 = R_TOTAL - (wid + 1) * HALF
    pltpu.sync_copy(thrk.at[pl.ds(0, HALF)], kthr_hbm.at[pl.ds(a0, HALF)])
    pltpu.sync_copy(thrj.at[pl.ds(0, HALF)], jthr_hbm.at[pl.ds(a0, HALF)])
    pltpu.sync_copy(thrk.at[pl.ds(HALF, HALF)], kthr_hbm.at[pl.ds(b0, HALF)])
    pltpu.sync_copy(thrj.at[pl.ds(HALF, HALF)], jthr_hbm.at[pl.ds(b0, HALF)])


def _sc_topk(x):
    mesh = plsc.VectorSubcoreMesh(core_axis_name="c", subcore_axis_name="s")
    kern = functools.partial(
        pl.kernel,
        out_type=(
            jax.ShapeDtypeStruct((R_TOTAL, K_TOP), jnp.int32),
            jax.ShapeDtypeStruct((R_TOTAL,), jnp.int32),
            jax.ShapeDtypeStruct((R_TOTAL,), jnp.int32),
        ),
        mesh=mesh,
        compiler_params=pltpu.CompilerParams(needs_layout_passes=False),
        scratch_types=[
            pltpu.VMEM((N_DIM,), jnp.float32),
            pltpu.VMEM((N_DIM,), jnp.float32),
            pltpu.VMEM((N_DIM,), jnp.int32),
            pltpu.VMEM((N_DIM,), jnp.int32),
            pltpu.VMEM((N_DIM,), jnp.int32),
            pltpu.VMEM((N_DIM,), jnp.int32),
            pltpu.VMEM((1024,), jnp.int32),
            pltpu.VMEM((1024,), jnp.int32),
            pltpu.VMEM((256,), jnp.int32),
            pltpu.VMEM((256,), jnp.int32),
            pltpu.VMEM((256,), jnp.int32),
            pltpu.VMEM((256,), jnp.int32),
            pltpu.VMEM((2 * HALF,), jnp.int32),
            pltpu.VMEM((2 * HALF,), jnp.int32),
            pltpu.SemaphoreType.DMA,
            pltpu.SemaphoreType.DMA,
            pltpu.SemaphoreType.DMA,
        ],
    )(_sc_sort_kernel)
    return kern(x)


def _mask_body(x_ref, kthr_ref, jthr_ref, o_ref):
    qb = pl.program_id(0)
    rows = x_ref.shape[0]
    s = x_ref[...]
    bv = lax.bitcast_convert_type(s, jnp.int32)
    key = jnp.where(bv >= 0, bv, INT_MIN - bv)
    col = lax.broadcasted_iota(jnp.int32, s.shape, 1)
    q0 = (qb * rows) % Q_DIM
    rowq = lax.broadcasted_iota(jnp.int32, s.shape, 0) + q0
    key = jnp.where(col > rowq, INT_MIN, key)
    kthr = jnp.broadcast_to(kthr_ref[:, 0:1], s.shape)
    jthr = jnp.broadcast_to(jthr_ref[:, 0:1], s.shape)
    o_ref[...] = (key > kthr) | ((key == kthr) & (col <= jthr))


def _tc_mask(x, kthr, jthr):
    rows = 256
    grid = (R_TOTAL // rows,)
    return pl.pallas_call(
        _mask_body,
        grid=grid,
        in_specs=[
            pl.BlockSpec((rows, N_DIM), lambda i: (i, 0)),
            pl.BlockSpec((rows, 8), lambda i: (i, 0)),
            pl.BlockSpec((rows, 8), lambda i: (i, 0)),
        ],
        out_specs=pl.BlockSpec((rows, N_DIM), lambda i: (i, 0)),
        out_shape=jax.ShapeDtypeStruct((R_TOTAL, N_DIM), jnp.bool_),
    )(x, kthr, jthr)


def kernel(index_scores):
    x = index_scores.reshape(R_TOTAL, N_DIM)
    idx, kthr, jthr = _sc_topk(x)
    kthr8 = jnp.broadcast_to(kthr[:, None], (R_TOTAL, 8))
    jthr8 = jnp.broadcast_to(jthr[:, None], (R_TOTAL, 8))
    mask = _tc_mask(x, kthr8, jthr8)
    return (mask.reshape(B_DIM, Q_DIM, N_DIM),
            idx.reshape(B_DIM, Q_DIM, K_TOP))


# speedup vs baseline: 1.2683x; 1.2683x over previous
import functools

import numpy as np
import jax
import jax.numpy as jnp
from jax import lax
from jax.experimental import pallas as pl
from jax.experimental.pallas import tpu as pltpu
from jax.experimental.pallas import tpu_sc as plsc

L = 16
B_DIM, Q_DIM, N_DIM = 2, 4096, 4096
R_TOTAL = B_DIM * Q_DIM
K_TOP = 2048
NUM_WORKERS = 32
HALF = R_TOTAL // (2 * NUM_WORKERS)
INT_MIN = np.int32(-2147483648)
ALL_ONES = np.int32(-1)


def _splat(x):
    return jnp.full((L,), x, jnp.int32)


def _sc_sort_kernel(x_hbm, idx_hbm, kthr_hbm, jthr_hbm,
                    xb0, xb1, ka, ia, kb, ib, h0, h1, thrk, thrj,
                    sem0, sem1, semo):
    wid = lax.axis_index("s") * 2 + lax.axis_index("c")
    lanes = lax.iota(jnp.int32, L)
    ones = _splat(1)

    def scan_hist(h):
        def chunk(t, carry):
            hv = h[pl.ds(t * L, L)]
            incl = plsc.cumsum(hv)
            h[pl.ds(t * L, L)] = incl - hv + carry
            return carry + incl[L - 1]
        lax.fori_loop(0, 256 // L, chunk, jnp.int32(-1))

    def zero_hist(h):
        def chunk(t, c):
            h[pl.ds(t * L, L)] = jnp.zeros((L,), jnp.int32)
            return c
        lax.fori_loop(0, 256 // L, chunk, jnp.int32(0))

    def radix_pass(v, shift, ksrc, isrc, kdst, idst, run, hnext, next_shift):
        def body(t, c):
            kv = ksrc[pl.ds(t * L, L)]
            if shift == 0:
                d = lax.bitwise_and(kv, _splat(255))
            elif shift == 24:
                d = lax.shift_right_logical(kv, _splat(24))
            else:
                d = lax.bitwise_and(
                    lax.shift_right_logical(kv, _splat(shift)), _splat(255))
            occ, _ = plsc.scan_count(d)
            base = plsc.load_gather(run, [d])
            pos = base + occ
            if isrc is None:
                iv = lanes + t * L
            else:
                iv = isrc[pl.ds(t * L, L)]
            plsc.store_scatter(kdst, [pos], kv)
            plsc.store_scatter(idst, [pos], iv)
            plsc.store_scatter(run, [d], pos)
            if hnext is not None:
                if next_shift == 24:
                    d2 = lax.shift_right_logical(kv, _splat(24))
                else:
                    d2 = lax.bitwise_and(
                        lax.shift_right_logical(kv, _splat(next_shift)),
                        _splat(255))
                plsc.addupdate_scatter(hnext, [d2], ones)
            return c
        lax.fori_loop(0, v, body, jnp.int32(0))

    def row_of(g):
        row_a = wid * HALF + g
        row_b = R_TOTAL - 1 - wid * HALF - (g - HALF)
        return jnp.where(g < HALF, row_a, row_b)

    def row_body(i, xb, sem_self, xb_other, sem_other):
        row = row_of(i)
        q = lax.rem(row, jnp.int32(Q_DIM))
        v = (q + L) // L

        nxt = row_of(lax.min(i + 1, jnp.int32(2 * HALF - 1)))
        pltpu.async_copy(x_hbm.at[nxt], xb_other, sem_other)
        pltpu.make_async_copy(x_hbm.at[row], xb, sem_self).wait()

        zero_hist(h0)

        def keys_of(t):
            xv = xb[pl.ds(t * L, L)]
            bv = plsc.bitcast(xv, jnp.int32)
            key = jnp.where(bv >= 0, bv, INT_MIN - bv)
            return lax.bitwise_xor(key, _splat(0x7FFFFFFF))

        def emit(t, inv):
            ka[pl.ds(t * L, L)] = inv
            d0 = lax.bitwise_and(inv, _splat(255))
            plsc.addupdate_scatter(h0, [d0], ones)

        def build(t, c2):
            emit(t, keys_of(t))
            return c2
        lax.fori_loop(0, v - 1, build, jnp.int32(0))
        tl = v - 1
        inv_l = jnp.where(lanes + tl * L <= q, keys_of(tl), _splat(ALL_ONES))
        emit(tl, inv_l)

        scan_hist(h0)
        zero_hist(h1)
        radix_pass(v, 0, ka, None, kb, ib, h0, h1, 8)
        pltpu.make_async_copy(ia.at[pl.ds(0, K_TOP)], idx_hbm.at[row],
                              semo).wait()
        scan_hist(h1)
        zero_hist(h0)
        radix_pass(v, 8, kb, ib, ka, ia, h1, h0, 16)
        scan_hist(h0)
        zero_hist(h1)
        radix_pass(v, 16, ka, ia, kb, ib, h0, h1, 24)
        scan_hist(h1)
        radix_pass(v, 24, kb, ib, ka, ia, h1, None, 0)

        def fill(t, c2):
            ia[pl.ds(t * L, L)] = lanes + t * L
            return c2
        lax.fori_loop(lax.min(v, jnp.int32(K_TOP // L)),
                      jnp.int32(K_TOP // L), fill, jnp.int32(0))

        has = (v * L) >= K_TOP
        tvk = ka[pl.ds(K_TOP - L, L)]
        tvj = ia[pl.ds(K_TOP - L, L)]
        ithr = jnp.where(has, tvk[L - 1], ALL_ONES)
        jthr = jnp.where(has, tvj[L - 1], jnp.int32(K_TOP - 1))
        kthr = lax.bitwise_xor(ithr, jnp.int32(0x7FFFFFFF))
        pos = _splat(jnp.where(i < HALF, i, 3 * HALF - 1 - i))
        plsc.store_scatter(thrk, [pos], _splat(kthr))
        plsc.store_scatter(thrj, [pos], _splat(jthr))

        pltpu.async_copy(ia.at[pl.ds(0, K_TOP)], idx_hbm.at[row], semo)

    r0 = row_of(jnp.int32(0))
    pltpu.async_copy(x_hbm.at[r0], xb0, sem0)
    pltpu.async_copy(ia.at[pl.ds(0, K_TOP)], idx_hbm.at[r0], semo)

    def pair_body(p, c):
        row_body(2 * p, xb0, sem0, xb1, sem1)
        row_body(2 * p + 1, xb1, sem1, xb0, sem0)
        return c

    lax.fori_loop(0, HALF, pair_body, jnp.int32(0))

    pltpu.make_async_copy(ia.at[pl.ds(0, K_TOP)], idx_hbm.at[r0], semo).wait()
    pltpu.make_async_copy(x_hbm.at[r0], xb0, sem0).wait()

    a0 = wid * HALF
    b0 = R_TOTAL - (wid + 1) * HALF
    pltpu.sync_copy(thrk.at[pl.ds(0, HALF)], kthr_hbm.at[pl.ds(a0, HALF)])
    pltpu.sync_copy(thrj.at[pl.ds(0, HALF)], jthr_hbm.at[pl.ds(a0, HALF)])
    pltpu.sync_copy(thrk.at[pl.ds(HALF, HALF)], kthr_hbm.at[pl.ds(b0, HALF)])
    pltpu.sync_copy(thrj.at[pl.ds(HALF, HALF)], jthr_hbm.at[pl.ds(b0, HALF)])


def _sc_topk(x):
    mesh = plsc.VectorSubcoreMesh(core_axis_name="c", subcore_axis_name="s")
    kern = functools.partial(
        pl.kernel,
        out_type=(
            jax.ShapeDtypeStruct((R_TOTAL, K_TOP), jnp.int32),
            jax.ShapeDtypeStruct((R_TOTAL,), jnp.int32),
            jax.ShapeDtypeStruct((R_TOTAL,), jnp.int32),
        ),
        mesh=mesh,
        compiler_params=pltpu.CompilerParams(needs_layout_passes=False),
        scratch_types=[
            pltpu.VMEM((N_DIM,), jnp.float32),
            pltpu.VMEM((N_DIM,), jnp.float32),
            pltpu.VMEM((N_DIM,), jnp.int32),
            pltpu.VMEM((N_DIM,), jnp.int32),
            pltpu.VMEM((N_DIM,), jnp.int32),
            pltpu.VMEM((N_DIM,), jnp.int32),
            pltpu.VMEM((256,), jnp.int32),
            pltpu.VMEM((256,), jnp.int32),
            pltpu.VMEM((2 * HALF,), jnp.int32),
            pltpu.VMEM((2 * HALF,), jnp.int32),
            pltpu.SemaphoreType.DMA,
            pltpu.SemaphoreType.DMA,
            pltpu.SemaphoreType.DMA,
        ],
    )(_sc_sort_kernel)
    return kern(x)


def _mask_body(x_ref, kthr_ref, jthr_ref, o_ref):
    qb = pl.program_id(0)
    rows = x_ref.shape[0]
    s = x_ref[...]
    bv = lax.bitcast_convert_type(s, jnp.int32)
    key = jnp.where(bv >= 0, bv, INT_MIN - bv)
    col = lax.broadcasted_iota(jnp.int32, s.shape, 1)
    q0 = (qb * rows) % Q_DIM
    rowq = lax.broadcasted_iota(jnp.int32, s.shape, 0) + q0
    key = jnp.where(col > rowq, INT_MIN, key)
    kthr = jnp.broadcast_to(kthr_ref[:, 0:1], s.shape)
    jthr = jnp.broadcast_to(jthr_ref[:, 0:1], s.shape)
    o_ref[...] = (key > kthr) | ((key == kthr) & (col <= jthr))


def _tc_mask(x, kthr, jthr):
    rows = 256
    grid = (R_TOTAL // rows,)
    return pl.pallas_call(
        _mask_body,
        grid=grid,
        in_specs=[
            pl.BlockSpec((rows, N_DIM), lambda i: (i, 0)),
            pl.BlockSpec((rows, 8), lambda i: (i, 0)),
            pl.BlockSpec((rows, 8), lambda i: (i, 0)),
        ],
        out_specs=pl.BlockSpec((rows, N_DIM), lambda i: (i, 0)),
        out_shape=jax.ShapeDtypeStruct((R_TOTAL, N_DIM), jnp.bool_),
    )(x, kthr, jthr)


def kernel(index_scores):
    x = index_scores.reshape(R_TOTAL, N_DIM)
    idx, kthr, jthr = _sc_topk(x)
    kthr8 = jnp.broadcast_to(kthr[:, None], (R_TOTAL, 8))
    jthr8 = jnp.broadcast_to(jthr[:, None], (R_TOTAL, 8))
    mask = _tc_mask(x, kthr8, jthr8)
    return (mask.reshape(B_DIM, Q_DIM, N_DIM),
            idx.reshape(B_DIM, Q_DIM, K_TOP))


# speedup vs baseline: 1.2780x; 1.0077x over previous
import functools

import numpy as np
import jax
import jax.numpy as jnp
from jax import lax
from jax.experimental import pallas as pl
from jax.experimental.pallas import tpu as pltpu
from jax.experimental.pallas import tpu_sc as plsc

L = 16
B_DIM, Q_DIM, N_DIM = 2, 4096, 4096
R_TOTAL = B_DIM * Q_DIM
K_TOP = 2048
NUM_WORKERS = 32
HALF = R_TOTAL // (2 * NUM_WORKERS)
INT_MIN = np.int32(-2147483648)
ALL_ONES = np.int32(-1)


def _splat(x):
    return jnp.full((L,), x, jnp.int32)


def _sc_sort_kernel(x_hbm, idx_hbm, kthr_hbm, jthr_hbm,
                    xb0, xb1, ka, ia, kb, ib, k0, h0, h1, thrk, thrj,
                    sem0, sem1, semo):
    wid = lax.axis_index("s") * 2 + lax.axis_index("c")
    lanes = lax.iota(jnp.int32, L)
    ones = _splat(1)

    def scan_hist(h):
        def chunk(t, carry):
            hv = h[pl.ds(t * L, L)]
            incl = plsc.cumsum(hv)
            h[pl.ds(t * L, L)] = incl - hv + carry
            return carry + incl[L - 1]
        lax.fori_loop(0, 256 // L, chunk, jnp.int32(-1))

    def zero_hist(h):
        def chunk(t, c):
            h[pl.ds(t * L, L)] = jnp.zeros((L,), jnp.int32)
            return c
        lax.fori_loop(0, 256 // L, chunk, jnp.int32(0))

    def radix_pass(v, shift, ksrc, isrc, kdst, idst, run, hnext, next_shift):
        def body(t, c):
            kv = ksrc[pl.ds(t * L, L)]
            if shift == 0:
                d = lax.bitwise_and(kv, _splat(255))
            elif shift == 24:
                d = lax.shift_right_logical(kv, _splat(24))
            else:
                d = lax.bitwise_and(
                    lax.shift_right_logical(kv, _splat(shift)), _splat(255))
            occ, _ = plsc.scan_count(d)
            base = plsc.load_gather(run, [d])
            pos = base + occ
            if isrc is None:
                iv = lanes + t * L
            else:
                iv = isrc[pl.ds(t * L, L)]
            if kdst is not None:
                plsc.store_scatter(kdst, [pos], kv)
            plsc.store_scatter(idst, [pos], iv)
            plsc.store_scatter(run, [d], pos)
            if hnext is not None:
                if next_shift == 24:
                    d2 = lax.shift_right_logical(kv, _splat(24))
                else:
                    d2 = lax.bitwise_and(
                        lax.shift_right_logical(kv, _splat(next_shift)),
                        _splat(255))
                plsc.addupdate_scatter(hnext, [d2], ones)
            return c
        lax.fori_loop(0, v, body, jnp.int32(0))

    def row_of(g):
        row_a = wid * HALF + g
        row_b = R_TOTAL - 1 - wid * HALF - (g - HALF)
        return jnp.where(g < HALF, row_a, row_b)

    def row_body(i, xb, sem_self, xb_other, sem_other):
        row = row_of(i)
        q = lax.rem(row, jnp.int32(Q_DIM))
        v = (q + L) // L

        nxt = row_of(lax.min(i + 1, jnp.int32(2 * HALF - 1)))
        pltpu.async_copy(x_hbm.at[nxt], xb_other, sem_other)
        pltpu.make_async_copy(x_hbm.at[row], xb, sem_self).wait()

        zero_hist(h0)

        def keys_of(t):
            xv = xb[pl.ds(t * L, L)]
            bv = plsc.bitcast(xv, jnp.int32)
            key = jnp.where(bv >= 0, bv, INT_MIN - bv)
            return lax.bitwise_xor(key, _splat(0x7FFFFFFF))

        def emit(t, inv):
            ka[pl.ds(t * L, L)] = inv
            k0[pl.ds(t * L, L)] = inv
            d0 = lax.bitwise_and(inv, _splat(255))
            plsc.addupdate_scatter(h0, [d0], ones)

        def build(t, c2):
            emit(t, keys_of(t))
            return c2
        lax.fori_loop(0, v - 1, build, jnp.int32(0))
        tl = v - 1
        inv_l = jnp.where(lanes + tl * L <= q, keys_of(tl), _splat(ALL_ONES))
        emit(tl, inv_l)

        scan_hist(h0)
        zero_hist(h1)
        radix_pass(v, 0, ka, None, kb, ib, h0, h1, 8)
        pltpu.make_async_copy(ia.at[pl.ds(0, K_TOP)], idx_hbm.at[row],
                              semo).wait()
        scan_hist(h1)
        zero_hist(h0)
        radix_pass(v, 8, kb, ib, ka, ia, h1, h0, 16)
        scan_hist(h0)
        zero_hist(h1)
        radix_pass(v, 16, ka, ia, kb, ib, h0, h1, 24)
        scan_hist(h1)
        radix_pass(v, 24, kb, ib, None, ia, h1, None, 0)

        def fill(t, c2):
            ia[pl.ds(t * L, L)] = lanes + t * L
            return c2
        lax.fori_loop(lax.min(v, jnp.int32(K_TOP // L)),
                      jnp.int32(K_TOP // L), fill, jnp.int32(0))

        has = (v * L) >= K_TOP
        tvj = ia[pl.ds(K_TOP - L, L)]
        jthr = jnp.where(has, tvj[L - 1], jnp.int32(K_TOP - 1))
        tvk = plsc.load_gather(k0, [_splat(jthr)])
        ithr = jnp.where(has, tvk[L - 1], ALL_ONES)
        kthr = lax.bitwise_xor(ithr, jnp.int32(0x7FFFFFFF))
        pos = _splat(jnp.where(i < HALF, i, 3 * HALF - 1 - i))
        plsc.store_scatter(thrk, [pos], _splat(kthr))
        plsc.store_scatter(thrj, [pos], _splat(jthr))

        pltpu.async_copy(ia.at[pl.ds(0, K_TOP)], idx_hbm.at[row], semo)

    r0 = row_of(jnp.int32(0))
    pltpu.async_copy(x_hbm.at[r0], xb0, sem0)
    pltpu.async_copy(ia.at[pl.ds(0, K_TOP)], idx_hbm.at[r0], semo)

    def pair_body(p, c):
        row_body(2 * p, xb0, sem0, xb1, sem1)
        row_body(2 * p + 1, xb1, sem1, xb0, sem0)
        return c

    lax.fori_loop(0, HALF, pair_body, jnp.int32(0))

    pltpu.make_async_copy(ia.at[pl.ds(0, K_TOP)], idx_hbm.at[r0], semo).wait()
    pltpu.make_async_copy(x_hbm.at[r0], xb0, sem0).wait()

    a0 = wid * HALF
    b0 = R_TOTAL - (wid + 1) * HALF
    pltpu.sync_copy(thrk.at[pl.ds(0, HALF)], kthr_hbm.at[pl.ds(a0, HALF)])
    pltpu.sync_copy(thrj.at[pl.ds(0, HALF)], jthr_hbm.at[pl.ds(a0, HALF)])
    pltpu.sync_copy(thrk.at[pl.ds(HALF, HALF)], kthr_hbm.at[pl.ds(b0, HALF)])
    pltpu.sync_copy(thrj.at[pl.ds(HALF, HALF)], jthr_hbm.at[pl.ds(b0, HALF)])


def _sc_topk(x):
    mesh = plsc.VectorSubcoreMesh(core_axis_name="c", subcore_axis_name="s")
    kern = functools.partial(
        pl.kernel,
        out_type=(
            jax.ShapeDtypeStruct((R_TOTAL, K_TOP), jnp.int32),
            jax.ShapeDtypeStruct((R_TOTAL,), jnp.int32),
            jax.ShapeDtypeStruct((R_TOTAL,), jnp.int32),
        ),
        mesh=mesh,
        compiler_params=pltpu.CompilerParams(needs_layout_passes=False),
        scratch_types=[
            pltpu.VMEM((N_DIM,), jnp.float32),
            pltpu.VMEM((N_DIM,), jnp.float32),
            pltpu.VMEM((N_DIM,), jnp.int32),
            pltpu.VMEM((N_DIM,), jnp.int32),
            pltpu.VMEM((N_DIM,), jnp.int32),
            pltpu.VMEM((N_DIM,), jnp.int32),
            pltpu.VMEM((N_DIM,), jnp.int32),
            pltpu.VMEM((256,), jnp.int32),
            pltpu.VMEM((256,), jnp.int32),
            pltpu.VMEM((2 * HALF,), jnp.int32),
            pltpu.VMEM((2 * HALF,), jnp.int32),
            pltpu.SemaphoreType.DMA,
            pltpu.SemaphoreType.DMA,
            pltpu.SemaphoreType.DMA,
        ],
    )(_sc_sort_kernel)
    return kern(x)


def _mask_body(x_ref, kthr_ref, jthr_ref, o_ref):
    qb = pl.program_id(0)
    rows = x_ref.shape[0]
    s = x_ref[...]
    bv = lax.bitcast_convert_type(s, jnp.int32)
    key = jnp.where(bv >= 0, bv, INT_MIN - bv)
    col = lax.broadcasted_iota(jnp.int32, s.shape, 1)
    q0 = (qb * rows) % Q_DIM
    rowq = lax.broadcasted_iota(jnp.int32, s.shape, 0) + q0
    key = jnp.where(col > rowq, INT_MIN, key)
    kthr = jnp.broadcast_to(kthr_ref[:, 0:1], s.shape)
    jthr = jnp.broadcast_to(jthr_ref[:, 0:1], s.shape)
    o_ref[...] = (key > kthr) | ((key == kthr) & (col <= jthr))


def _tc_mask(x, kthr, jthr):
    rows = 256
    grid = (R_TOTAL // rows,)
    return pl.pallas_call(
        _mask_body,
        grid=grid,
        in_specs=[
            pl.BlockSpec((rows, N_DIM), lambda i: (i, 0)),
            pl.BlockSpec((rows, 8), lambda i: (i, 0)),
            pl.BlockSpec((rows, 8), lambda i: (i, 0)),
        ],
        out_specs=pl.BlockSpec((rows, N_DIM), lambda i: (i, 0)),
        out_shape=jax.ShapeDtypeStruct((R_TOTAL, N_DIM), jnp.bool_),
    )(x, kthr, jthr)


def kernel(index_scores):
    x = index_scores.reshape(R_TOTAL, N_DIM)
    idx, kthr, jthr = _sc_topk(x)
    kthr8 = jnp.broadcast_to(kthr[:, None], (R_TOTAL, 8))
    jthr8 = jnp.broadcast_to(jthr[:, None], (R_TOTAL, 8))
    mask = _tc_mask(x, kthr8, jthr8)
    return (mask.reshape(B_DIM, Q_DIM, N_DIM),
            idx.reshape(B_DIM, Q_DIM, K_TOP))
